# baseline (speedup 1.0000x reference)
import functools

import jax
import jax.numpy as jnp
from jax import lax
from jax.experimental import pallas as pl
from jax.experimental.pallas import tpu as pltpu
from jax.experimental.pallas import tpu_sc as plsc

B = 16384
F = 26
D = 16
VOCAB = 1000000

NC = 2
NS = 16
NW = NC * NS
ROWS = B // NW
IDX_PER_W = ROWS * F
IDX_ROWS = IDX_PER_W // 128
E_CHUNK = 64
N_CHUNKS = ROWS // E_CHUNK


def _fm_sc_body(idx_hbm, e_hbm, w_hbm, fo_hbm, so_hbm, idx_v, wv_v, e_v, so_v, gsem):
    c = lax.axis_index("c")
    s = lax.axis_index("s")
    wid = s * NC + c
    row0 = wid * ROWS

    pltpu.sync_copy(idx_hbm.at[wid], idx_v)

    def fire(j, carry):
        pltpu.async_copy(w_hbm.at[idx_v.at[j]], wv_v.at[j], gsem)
        return carry

    lax.fori_loop(0, IDX_ROWS, fire, 0)

    for cc in range(N_CHUNKS):
        pltpu.sync_copy(e_hbm.at[pl.ds(row0 + cc * E_CHUNK, E_CHUNK)], e_v)

        def rowfn(r, carry):
            acc = jnp.zeros((D,), jnp.float32)
            sq = jnp.zeros((D,), jnp.float32)
            for f in range(F):
                v = e_v[r, pl.ds(f * D, D)]
                acc = acc + v
                sq = sq + v * v
            so_v[cc * E_CHUNK + r, :] = 0.5 * (acc * acc - sq)
            return carry

        lax.fori_loop(0, E_CHUNK, rowfn, 0)

    def drain(j, carry):
        pltpu.make_async_copy(w_hbm.at[pl.ds(0, 128)], wv_v.at[j], gsem).wait()
        return carry

    lax.fori_loop(0, IDX_ROWS, drain, 0)

    pltpu.sync_copy(wv_v, fo_hbm.at[wid])
    pltpu.sync_copy(so_v, so_hbm.at[pl.ds(row0, ROWS)])


@functools.cache
def _fm_sc():
    return pl.kernel(
        _fm_sc_body,
        out_type=(
            jax.ShapeDtypeStruct((NW, IDX_ROWS, 128), jnp.float32),
            jax.ShapeDtypeStruct((B, D), jnp.float32),
        ),
        mesh=plsc.VectorSubcoreMesh(
            core_axis_name="c", subcore_axis_name="s",
            num_cores=NC, num_subcores=NS,
        ),
        scratch_types=[
            pltpu.VMEM((IDX_ROWS, 128), jnp.int32),
            pltpu.VMEM((IDX_ROWS, 128), jnp.float32),
            pltpu.VMEM((E_CHUNK, F * D), jnp.float32),
            pltpu.VMEM((ROWS, D), jnp.float32),
            pltpu.SemaphoreType.DMA,
        ],
    )


def kernel(sparse_inputs, embed_inputs, w):
    idx = sparse_inputs.reshape(NW, IDX_ROWS, 128)
    e = embed_inputs.reshape(B, F * D)
    wf = w.reshape(VOCAB)
    fo, so = _fm_sc()(idx, e, wf)
    return jnp.concatenate([fo.reshape(B, F), so], axis=-1)


# speedup vs baseline: 2.1120x; 2.1120x over previous
import functools

import jax
import jax.numpy as jnp
from jax import lax
from jax.experimental import pallas as pl
from jax.experimental.pallas import tpu as pltpu
from jax.experimental.pallas import tpu_sc as plsc

B = 16384
F = 26
D = 16
VOCAB = 1000000

NC = 2
NS = 16
NW = NC * NS
COLS = B // NW
GATHERS = F * COLS // 128


def _gather_body(idx_hbm, w_hbm, fo_hbm, idx_v, wv_v, gsem):
    c = lax.axis_index("c")
    s = lax.axis_index("s")
    wid = s * NC + c
    col0 = wid * COLS

    pltpu.sync_copy(idx_hbm.at[:, pl.ds(col0, COLS)], idx_v)

    def fire(j, carry):
        f = j // (COLS // 128)
        k = j % (COLS // 128)
        pltpu.async_copy(
            w_hbm.at[idx_v.at[f, pl.ds(k * 128, 128)]],
            wv_v.at[f, pl.ds(k * 128, 128)],
            gsem,
        )
        return carry

    lax.fori_loop(0, GATHERS, fire, 0)

    def drain(j, carry):
        pltpu.make_async_copy(
            w_hbm.at[pl.ds(0, 128)], wv_v.at[0, pl.ds(0, 128)], gsem
        ).wait()
        return carry

    lax.fori_loop(0, GATHERS, drain, 0)

    pltpu.sync_copy(wv_v, fo_hbm.at[:, pl.ds(col0, COLS)])


@functools.cache
def _gather_sc():
    return pl.kernel(
        _gather_body,
        out_type=jax.ShapeDtypeStruct((F, B), jnp.float32),
        mesh=plsc.VectorSubcoreMesh(
            core_axis_name="c", subcore_axis_name="s",
            num_cores=NC, num_subcores=NS,
        ),
        scratch_types=[
            pltpu.VMEM((F, COLS), jnp.int32),
            pltpu.VMEM((F, COLS), jnp.float32),
            pltpu.SemaphoreType.DMA,
        ],
    )


def _pool_body(e_ref, o_ref):
    e = e_ref[...]
    ssum = jnp.sum(e, axis=0)
    ssq = jnp.sum(e * e, axis=0)
    o_ref[...] = 0.5 * (ssum * ssum - ssq)


def _pool_tc(eT):
    blk = 2048
    return pl.pallas_call(
        _pool_body,
        grid=(B // blk,),
        in_specs=[pl.BlockSpec((F, D, blk), lambda i: (0, 0, i))],
        out_specs=pl.BlockSpec((D, blk), lambda i: (0, i)),
        out_shape=jax.ShapeDtypeStruct((D, B), jnp.float32),
    )(eT)


def kernel(sparse_inputs, embed_inputs, w):
    idxT = sparse_inputs.T
    eT = embed_inputs.transpose(1, 2, 0)
    wf = w[:, 0]
    foT = _gather_sc()(idxT, wf)
    soT = _pool_tc(eT)
    return jnp.concatenate([foT, soT], axis=0).T
